# Initial kernel scaffold; baseline (speedup 1.0000x reference)
import jax
import jax.numpy as jnp
from jax.experimental import pallas as pl

N = 10000
L = 32
NFEAT = 256
NHID = 512
NCLASS = 128

_M_BLK = 1250


def _mm1_body(x_ref, w_ref, o_ref):
    o_ref[...] = jnp.dot(x_ref[...], w_ref[...],
                         preferred_element_type=jnp.float32) * (1.0 / L)


def _mm2_body(x_ref, b_ref, w_ref, o_ref):
    h = jnp.maximum(x_ref[...] + b_ref[...], 0.0)
    o_ref[...] = jnp.dot(h, w_ref[...], preferred_element_type=jnp.float32)


def _mm1(x, w):
    m, k = x.shape
    n = w.shape[1]
    grid = m // _M_BLK
    return pl.pallas_call(
        _mm1_body,
        grid=(grid,),
        in_specs=[
            pl.BlockSpec((_M_BLK, k), lambda i: (i, 0)),
            pl.BlockSpec((k, n), lambda i: (0, 0)),
        ],
        out_specs=pl.BlockSpec((_M_BLK, n), lambda i: (i, 0)),
        out_shape=jax.ShapeDtypeStruct((m, n), jnp.float32),
    )(x, w)


def _mm2(x, b, w):
    m, k = x.shape
    n = w.shape[1]
    grid = m // _M_BLK
    return pl.pallas_call(
        _mm2_body,
        grid=(grid,),
        in_specs=[
            pl.BlockSpec((_M_BLK, k), lambda i: (i, 0)),
            pl.BlockSpec((1, k), lambda i: (0, 0)),
            pl.BlockSpec((k, n), lambda i: (0, 0)),
        ],
        out_specs=pl.BlockSpec((_M_BLK, n), lambda i: (i, 0)),
        out_shape=jax.ShapeDtypeStruct((m, n), jnp.float32),
    )(x, b.reshape(1, k), w)


def kernel(x_index, features_index, edge_index, edge_weight, embedding, W1, b1, W2, b2):
    src = edge_index[0]
    dst = edge_index[1]

    pooled = jnp.take(embedding, features_index.reshape(-1), axis=0,
                      mode="promise_in_bounds").reshape(N, L, NFEAT).sum(axis=1)
    support = _mm1(pooled, W1)

    msgs = edge_weight[:, None] * jnp.take(support, src, axis=0,
                                           mode="promise_in_bounds")
    agg1 = jax.ops.segment_sum(msgs, dst, num_segments=N)

    y = _mm2(agg1, b1, W2)

    msgs2 = edge_weight[:, None] * jnp.take(y, src, axis=0,
                                            mode="promise_in_bounds")
    agg2 = jax.ops.segment_sum(msgs2, dst, num_segments=N)

    return jnp.take(agg2, x_index, axis=0, mode="promise_in_bounds") + b2


# speedup vs baseline: 1.0193x; 1.0193x over previous
import jax
import jax.numpy as jnp
from jax.experimental import pallas as pl

N = 10000
L = 32
NFEAT = 256
NHID = 512
NCLASS = 128

_M_BLK = 1000


def _mm1_body(x_ref, w_ref, o_ref):
    o_ref[...] = jnp.dot(x_ref[...], w_ref[...],
                         preferred_element_type=jnp.float32) * (1.0 / L)


def _mm2_body(x_ref, b_ref, w_ref, o_ref):
    h = jnp.maximum(x_ref[...] + b_ref[...], 0.0)
    o_ref[...] = jnp.dot(h, w_ref[...], preferred_element_type=jnp.float32)


def _mm1(x, w):
    m, k = x.shape
    n = w.shape[1]
    grid = m // _M_BLK
    return pl.pallas_call(
        _mm1_body,
        grid=(grid,),
        in_specs=[
            pl.BlockSpec((_M_BLK, k), lambda i: (i, 0)),
            pl.BlockSpec((k, n), lambda i: (0, 0)),
        ],
        out_specs=pl.BlockSpec((_M_BLK, n), lambda i: (i, 0)),
        out_shape=jax.ShapeDtypeStruct((m, n), jnp.float32),
    )(x, w)


def _mm2(x, b, w):
    m, k = x.shape
    n = w.shape[1]
    grid = m // _M_BLK
    return pl.pallas_call(
        _mm2_body,
        grid=(grid,),
        in_specs=[
            pl.BlockSpec((_M_BLK, k), lambda i: (i, 0)),
            pl.BlockSpec((1, k), lambda i: (0, 0)),
            pl.BlockSpec((k, n), lambda i: (0, 0)),
        ],
        out_specs=pl.BlockSpec((_M_BLK, n), lambda i: (i, 0)),
        out_shape=jax.ShapeDtypeStruct((m, n), jnp.float32),
    )(x, b.reshape(1, k), w)


def kernel(x_index, features_index, edge_index, edge_weight, embedding, W1, b1, W2, b2):
    src = edge_index[0]
    dst = edge_index[1]

    pooled = jnp.take(embedding, features_index.reshape(-1), axis=0,
                      mode="clip").reshape(N, L, NFEAT).sum(axis=1)
    support = _mm1(pooled, W1)

    msgs = edge_weight[:, None] * jnp.take(support, src, axis=0,
                                           mode="clip")
    agg1 = jax.ops.segment_sum(msgs, dst, num_segments=N)

    y = _mm2(agg1, b1, W2)

    msgs2 = edge_weight[:, None] * jnp.take(y, src, axis=0,
                                            mode="clip")
    agg2 = jax.ops.segment_sum(msgs2, dst, num_segments=N)

    return jnp.take(agg2, x_index, axis=0, mode="clip") + b2


# speedup vs baseline: 2.3636x; 2.3189x over previous
import functools

import jax
import jax.numpy as jnp
from jax import lax
from jax.experimental import pallas as pl
from jax.experimental.pallas import tpu as pltpu
from jax.experimental.pallas import tpu_sc as plsc

N = 10000
E = 160000
L = 32
NFEAT = 256
NHID = 512
NCLASS = 128

NC, NS = 2, 16
NT = NC * NS
NP = 10112
EP = 163840
STRIPE = NP // NS

_M_BLK = 1264
_mesh = plsc.VectorSubcoreMesh(core_axis_name="c", subcore_axis_name="s")


def _spmm_body(nch, s_ref, src_ref, dst_ref, w_ref, out_ref,
               acc, srcb, dstb, wb, idxb, rows, obuf, sem):
    cid = lax.axis_index("c")
    sid = lax.axis_index("s")
    nwin = (EP // 128) // NS if nch == 4 else (EP // 128) // NT

    if nch == 4:
        erow0 = sid * nwin
    else:
        erow0 = (cid * NS + sid) * nwin
    pltpu.sync_copy(src_ref.at[pl.ds(erow0, nwin)], srcb)
    pltpu.sync_copy(dst_ref.at[pl.ds(erow0, nwin)], dstb)
    pltpu.sync_copy(w_ref.at[pl.ds(erow0 * 128, nwin * 128)], wb)

    stripe0 = sid * STRIPE
    zero16 = jnp.zeros((16,), jnp.float32)
    chunks_per_core = 2 if nch == 4 else 1

    for k in range(chunks_per_core):
        if nch == 4:
            chunk = cid * chunks_per_core + k
            off = chunk * NP
            orow_base = chunk * NP + stripe0
        else:
            off = 0
            orow_base = cid * NP + stripe0

        for r in range(8):
            for j in range(8):
                obuf[r, pl.ds(16 * j, 16)] = zero16

        def zloop(i, _):
            pltpu.sync_copy(obuf, acc.at[pl.ds(stripe0 + i * 8, 8)])
            return 0
        lax.fori_loop(0, STRIPE // 8, zloop, 0)
        plsc.subcore_barrier()

        def win(wi, _):
            if nch == 4:
                for j in range(8):
                    idxb[0, pl.ds(16 * j, 16)] = (
                        srcb[wi, pl.ds(16 * j, 16)] + off)
                gidx = idxb.at[0]
            else:
                gidx = srcb.at[wi]
            pltpu.async_copy(s_ref.at[gidx], rows, sem).wait()

            def grp(g, _):
                base = g * 16
                w16 = wb[pl.ds(wi * 128 + base, 16)]
                for e in range(16):
                    wv = jnp.full((16,), w16[e], jnp.float32)
                    for j in range(8):
                        rows[base + e, pl.ds(16 * j, 16)] = (
                            rows[base + e, pl.ds(16 * j, 16)] * wv)
                return 0
            lax.fori_loop(0, 8, grp, 0)

            pltpu.sync_copy(rows, acc.at[dstb.at[wi]], add=True)
            return 0
        lax.fori_loop(0, nwin, win, 0)
        plsc.subcore_barrier()

        def oloop(i, _):
            pltpu.sync_copy(acc.at[pl.ds(stripe0 + i * 8, 8)], obuf)
            pltpu.sync_copy(obuf, out_ref.at[pl.ds(orow_base + i * 8, 8)])
            return 0
        lax.fori_loop(0, STRIPE // 8, oloop, 0)
        plsc.subcore_barrier()


def _make_spmm(nch):
    nwin = (EP // 128) // NS if nch == 4 else (EP // 128) // NT
    nout = nch if nch == 4 else 2
    return pl.kernel(
        functools.partial(_spmm_body, nch),
        out_type=jax.ShapeDtypeStruct((nout * NP, 128), jnp.float32),
        mesh=_mesh,
        scratch_types=[
            pltpu.VMEM_SHARED((NP, 128), jnp.float32),
            pltpu.VMEM((nwin, 128), jnp.int32),
            pltpu.VMEM((nwin, 128), jnp.int32),
            pltpu.VMEM((nwin * 128,), jnp.float32),
            pltpu.VMEM((1, 128), jnp.int32),
            pltpu.VMEM((128, 128), jnp.float32),
            pltpu.VMEM((8, 128), jnp.float32),
            pltpu.SemaphoreType.DMA,
        ],
    )


_spmm4 = _make_spmm(4)
_spmm1c = _make_spmm(1)


def _mm1_body(x_ref, w_ref, o_ref):
    o_ref[0] = jnp.dot(x_ref[...], w_ref[...],
                       preferred_element_type=jnp.float32) * (1.0 / L)


def _mm1(x, w):
    return pl.pallas_call(
        _mm1_body,
        grid=(NP // _M_BLK, NHID // 128),
        in_specs=[
            pl.BlockSpec((_M_BLK, NFEAT), lambda i, c: (i, 0)),
            pl.BlockSpec((NFEAT, 128), lambda i, c: (0, c)),
        ],
        out_specs=pl.BlockSpec((1, _M_BLK, 128), lambda i, c: (c, i, 0)),
        out_shape=jax.ShapeDtypeStruct((NHID // 128, NP, 128), jnp.float32),
    )(x, w)


def _mm2_body(x_ref, b_ref, w_ref, o_ref):
    c = pl.program_id(1)
    h = jnp.maximum(x_ref[0] + b_ref[0], 0.0)
    p = jnp.dot(h, w_ref[0], preferred_element_type=jnp.float32)

    @pl.when(c == 0)
    def _():
        o_ref[...] = p

    @pl.when(c > 0)
    def _():
        o_ref[...] += p


def _mm2(x, b, w):
    return pl.pallas_call(
        _mm2_body,
        grid=(NP // _M_BLK, NHID // 128),
        in_specs=[
            pl.BlockSpec((1, _M_BLK, 128), lambda i, c: (c, i, 0)),
            pl.BlockSpec((1, 1, 128), lambda i, c: (c, 0, 0)),
            pl.BlockSpec((1, 128, NCLASS), lambda i, c: (c, 0, 0)),
        ],
        out_specs=pl.BlockSpec((_M_BLK, NCLASS), lambda i, c: (i, 0)),
        out_shape=jax.ShapeDtypeStruct((NP, NCLASS), jnp.float32),
    )(x, b.reshape(NHID // 128, 1, 128), w.reshape(NHID // 128, 128, NCLASS))


def kernel(x_index, features_index, edge_index, edge_weight, embedding, W1, b1, W2, b2):
    src = edge_index[0]
    dst = edge_index[1]
    pad = EP - E
    fill = (jnp.arange(pad, dtype=jnp.int32) % N)
    src2 = jnp.concatenate([src, fill]).reshape(EP // 128, 128)
    dst2 = jnp.concatenate([dst, fill]).reshape(EP // 128, 128)
    w2 = jnp.concatenate([edge_weight, jnp.zeros((pad,), jnp.float32)])

    pooled = jnp.take(embedding, features_index.reshape(-1), axis=0,
                      mode="clip").reshape(N, L, NFEAT).sum(axis=1)
    pooled = jnp.pad(pooled, ((0, NP - N), (0, 0)))

    support = _mm1(pooled, W1)
    agg1 = _spmm4(support.reshape(4 * NP, 128), src2, dst2, w2)

    y = _mm2(agg1.reshape(4, NP, 128), b1, W2)

    agg2 = _spmm1c(y, src2, dst2, w2)

    xi = x_index.astype(jnp.int32)
    return (jnp.take(agg2, xi, axis=0, mode="clip")
            + jnp.take(agg2, NP + xi, axis=0, mode="clip") + b2)


# speedup vs baseline: 4.2427x; 1.7951x over previous
import functools

import jax
import jax.numpy as jnp
from jax import lax
from jax.experimental import pallas as pl
from jax.experimental.pallas import tpu as pltpu
from jax.experimental.pallas import tpu_sc as plsc

N = 10000
E = 160000
L = 32
NFEAT = 256
NHID = 512
NCLASS = 128

NC, NS = 2, 16
NT = NC * NS
NP = 10240
EP = 163840
STRIPE = NP // NS

_M_BLK = 1280
_mesh = plsc.VectorSubcoreMesh(core_axis_name="c", subcore_axis_name="s")


def _spmm_body(nch, s_ref, src_ref, dst_ref, w_ref, out_ref,
               acc, srcb, dstb, wb, idxb, rows, obuf, sem):
    cid = lax.axis_index("c")
    sid = lax.axis_index("s")
    nwin = (EP // 128) // NS if nch == 4 else (EP // 128) // NT

    if nch == 4:
        erow0 = sid * nwin
    else:
        erow0 = (cid * NS + sid) * nwin
    pltpu.sync_copy(src_ref.at[pl.ds(erow0, nwin)], srcb)
    pltpu.sync_copy(dst_ref.at[pl.ds(erow0, nwin)], dstb)
    pltpu.sync_copy(w_ref.at[pl.ds(erow0 * 128, nwin * 128)], wb)

    stripe0 = sid * STRIPE
    zero16 = jnp.zeros((16,), jnp.float32)
    chunks_per_core = 2 if nch == 4 else 1

    for k in range(chunks_per_core):
        if nch == 4:
            chunk = cid * chunks_per_core + k
            off = chunk * NP
            orow_base = chunk * NP + stripe0
        else:
            off = 0
            orow_base = cid * NP + stripe0

        for r in range(8):
            for j in range(8):
                obuf[r, pl.ds(16 * j, 16)] = zero16

        def zloop(i, _):
            pltpu.sync_copy(obuf, acc.at[pl.ds(stripe0 + i * 8, 8)])
            return 0
        lax.fori_loop(0, STRIPE // 8, zloop, 0)
        plsc.subcore_barrier()

        def win(wi, _):
            if nch == 4:
                for j in range(8):
                    idxb[0, pl.ds(16 * j, 16)] = (
                        srcb[wi, pl.ds(16 * j, 16)] + off)
                gidx = idxb.at[0]
            else:
                gidx = srcb.at[wi]
            pltpu.async_copy(s_ref.at[gidx], rows, sem).wait()

            def grp(g, _):
                base = g * 16
                w16 = wb[pl.ds(wi * 128 + base, 16)]
                for e in range(16):
                    wv = jnp.full((16,), w16[e], jnp.float32)
                    for j in range(8):
                        rows[base + e, pl.ds(16 * j, 16)] = (
                            rows[base + e, pl.ds(16 * j, 16)] * wv)
                return 0
            lax.fori_loop(0, 8, grp, 0)

            pltpu.sync_copy(rows, acc.at[dstb.at[wi]], add=True)
            return 0
        lax.fori_loop(0, nwin, win, 0)
        plsc.subcore_barrier()

        def oloop(i, _):
            pltpu.sync_copy(acc.at[pl.ds(stripe0 + i * 8, 8)], obuf)
            pltpu.sync_copy(obuf, out_ref.at[pl.ds(orow_base + i * 8, 8)])
            return 0
        lax.fori_loop(0, STRIPE // 8, oloop, 0)
        plsc.subcore_barrier()


def _make_spmm(nch):
    nwin = (EP // 128) // NS if nch == 4 else (EP // 128) // NT
    nout = nch if nch == 4 else 2
    return pl.kernel(
        functools.partial(_spmm_body, nch),
        out_type=jax.ShapeDtypeStruct((nout * NP, 128), jnp.float32),
        mesh=_mesh,
        scratch_types=[
            pltpu.VMEM_SHARED((NP, 128), jnp.float32),
            pltpu.VMEM((nwin, 128), jnp.int32),
            pltpu.VMEM((nwin, 128), jnp.int32),
            pltpu.VMEM((nwin * 128,), jnp.float32),
            pltpu.VMEM((1, 128), jnp.int32),
            pltpu.VMEM((128, 128), jnp.float32),
            pltpu.VMEM((8, 128), jnp.float32),
            pltpu.SemaphoreType.DMA,
        ],
    )


_spmm4 = _make_spmm(4)
_spmm1c = _make_spmm(1)

_PNW = NP * L // 128 // NT
_PNODES = NP // NT


def _pool_sum(rows, base, sl):
    vs = [rows[base + r, sl] for r in range(L)]
    while len(vs) > 1:
        vs = [vs[i] + vs[i + 1] for i in range(0, len(vs) - 1, 2)] + (
            [vs[-1]] if len(vs) % 2 else [])
    return vs[0]


def _pool_body(fi_ref, emb_ref, out_ref, idxb, rows0, rows1, obuf,
               sem0, sem1):
    cid = lax.axis_index("c")
    sid = lax.axis_index("s")
    tid = cid * NS + sid
    pltpu.sync_copy(fi_ref.at[pl.ds(tid * _PNW, _PNW)], idxb)
    orow0 = tid * _PNODES

    def process(rows, half):
        def node(g, _):
            def col(j, _):
                sl = pl.ds(16 * j, 16)
                obuf[half + g, sl] = _pool_sum(rows, g * L, sl)
                return 0
            lax.fori_loop(0, NFEAT // 16, col, 0)
            return 0
        lax.fori_loop(0, 128 // L, node, 0)

    pltpu.async_copy(emb_ref.at[idxb.at[0]], rows0, sem0).wait()

    def step(k, _):
        w0 = k * 2
        pltpu.async_copy(emb_ref.at[idxb.at[w0 + 1]], rows1, sem1)
        process(rows0, 0)
        pltpu.make_async_copy(emb_ref.at[idxb.at[w0 + 1]], rows1,
                              sem1).wait()

        @pl.when(w0 + 2 < _PNW)
        def _():
            pltpu.async_copy(emb_ref.at[idxb.at[w0 + 2]], rows0, sem0)
        process(rows1, 4)

        pltpu.sync_copy(obuf, out_ref.at[pl.ds(orow0 + k * 8, 8)])

        @pl.when(w0 + 2 < _PNW)
        def _():
            pltpu.make_async_copy(emb_ref.at[idxb.at[w0 + 2]], rows0,
                                  sem0).wait()
        return 0
    lax.fori_loop(0, _PNW // 2, step, 0)


_pool = pl.kernel(
    _pool_body,
    out_type=jax.ShapeDtypeStruct((NP, NFEAT), jnp.float32),
    mesh=_mesh,
    scratch_types=[
        pltpu.VMEM((_PNW, 128), jnp.int32),
        pltpu.VMEM((128, NFEAT), jnp.float32),
        pltpu.VMEM((128, NFEAT), jnp.float32),
        pltpu.VMEM((8, NFEAT), jnp.float32),
        pltpu.SemaphoreType.DMA,
        pltpu.SemaphoreType.DMA,
    ],
)


def _mm1_body(x_ref, w_ref, o_ref):
    o_ref[0] = jnp.dot(x_ref[...], w_ref[...],
                       preferred_element_type=jnp.float32) * (1.0 / L)


def _mm1(x, w):
    return pl.pallas_call(
        _mm1_body,
        grid=(NP // _M_BLK, NHID // 128),
        in_specs=[
            pl.BlockSpec((_M_BLK, NFEAT), lambda i, c: (i, 0)),
            pl.BlockSpec((NFEAT, 128), lambda i, c: (0, c)),
        ],
        out_specs=pl.BlockSpec((1, _M_BLK, 128), lambda i, c: (c, i, 0)),
        out_shape=jax.ShapeDtypeStruct((NHID // 128, NP, 128), jnp.float32),
    )(x, w)


def _mm2_body(x_ref, b_ref, w_ref, o_ref):
    c = pl.program_id(1)
    h = jnp.maximum(x_ref[0] + b_ref[0], 0.0)
    p = jnp.dot(h, w_ref[0], preferred_element_type=jnp.float32)

    @pl.when(c == 0)
    def _():
        o_ref[...] = p

    @pl.when(c > 0)
    def _():
        o_ref[...] += p


def _mm2(x, b, w):
    return pl.pallas_call(
        _mm2_body,
        grid=(NP // _M_BLK, NHID // 128),
        in_specs=[
            pl.BlockSpec((1, _M_BLK, 128), lambda i, c: (c, i, 0)),
            pl.BlockSpec((1, 1, 128), lambda i, c: (c, 0, 0)),
            pl.BlockSpec((1, 128, NCLASS), lambda i, c: (c, 0, 0)),
        ],
        out_specs=pl.BlockSpec((_M_BLK, NCLASS), lambda i, c: (i, 0)),
        out_shape=jax.ShapeDtypeStruct((NP, NCLASS), jnp.float32),
    )(x, b.reshape(NHID // 128, 1, 128), w.reshape(NHID // 128, 128, NCLASS))


def kernel(x_index, features_index, edge_index, edge_weight, embedding, W1, b1, W2, b2):
    src = edge_index[0]
    dst = edge_index[1]
    pad = EP - E
    fill = (jnp.arange(pad, dtype=jnp.int32) % N)
    src2 = jnp.concatenate([src, fill]).reshape(EP // 128, 128)
    dst2 = jnp.concatenate([dst, fill]).reshape(EP // 128, 128)
    w2 = jnp.concatenate([edge_weight, jnp.zeros((pad,), jnp.float32)])

    fi_fill = (jnp.arange((NP - N) * L, dtype=jnp.int32)
               % embedding.shape[0]).reshape(NP - N, L)
    fi2 = jnp.concatenate([features_index, fi_fill]).reshape(NP * L // 128, 128)
    pooled = _pool(fi2, embedding)

    support = _mm1(pooled, W1)
    agg1 = _spmm4(support.reshape(4 * NP, 128), src2, dst2, w2)

    y = _mm2(agg1.reshape(4, NP, 128), b1, W2)

    agg2 = _spmm1c(y, src2, dst2, w2)

    xi = x_index.astype(jnp.int32)
    return (jnp.take(agg2, xi, axis=0, mode="clip")
            + jnp.take(agg2, NP + xi, axis=0, mode="clip") + b2)


# speedup vs baseline: 5.6060x; 1.3213x over previous
import functools

import jax
import jax.numpy as jnp
from jax import lax
from jax.experimental import pallas as pl
from jax.experimental.pallas import tpu as pltpu
from jax.experimental.pallas import tpu_sc as plsc

N = 10000
E = 160000
L = 32
NFEAT = 256
NHID = 512
NCLASS = 128

NC, NS = 2, 16
NT = NC * NS
NP = 10240
EP = 163840
STRIPE = NP // NS

_M_BLK = 1280
WE = 32
_mesh = plsc.VectorSubcoreMesh(core_axis_name="c", subcore_axis_name="s")


def _spmm_body(nch, s_ref, src_ref, dst_ref, w_ref, out_ref,
               acc, srcb, dstb, wring, rows0, rows1, obuf,
               g0, g1, ws0, ws1, s0, s1):
    cpc = 2 if nch == 4 else 1
    ept = EP // NS if nch == 4 else EP // NT
    nrows = ept // 128
    cid = lax.axis_index("c")
    sid = lax.axis_index("s")

    erow0 = sid * nrows if nch == 4 else (cid * NS + sid) * nrows
    pltpu.sync_copy(src_ref.at[pl.ds(erow0, nrows)], srcb)
    pltpu.sync_copy(dst_ref.at[pl.ds(erow0, nrows)], dstb)

    stripe0 = sid * STRIPE
    zero16 = jnp.zeros((16,), jnp.float32)

    def fetch_w(row, wslot, wsem):
        pltpu.async_copy(w_ref.at[erow0 + row], wring.at[wslot], wsem)

    def wait_w(wslot, wsem):
        pltpu.make_async_copy(w_ref.at[0], wring.at[wslot], wsem).wait()

    def gather(row, half, rows, gsem, offv):
        for g in range(4):
            sv = srcb[row, pl.ds(half * 64 + g * 16, 16)] + offv
            pltpu.async_copy(s_ref.at[sv], rows.at[pl.ds(g * 16, 16)], gsem)

    def wait_gather(rows, gsem):
        zv = jnp.zeros((16,), jnp.int32)
        for g in range(4):
            pltpu.make_async_copy(s_ref.at[zv], rows.at[pl.ds(0, 16)],
                                  gsem).wait()

    def scatter(row, half, rows, ssem):
        for g in range(4):
            dv = dstb[row, pl.ds(half * 64 + g * 16, 16)]
            pltpu.async_copy(rows.at[pl.ds(g * 16, 16)], acc.at[dv], ssem,
                             add=True)

    def wait_scatter(rows, ssem):
        zv = jnp.zeros((16,), jnp.int32)
        for g in range(4):
            pltpu.make_async_copy(rows.at[pl.ds(0, 16)], acc.at[zv],
                                  ssem).wait()

    def scale(rows, wslot, half):
        def grp(g, _):
            base = g * 16
            w16 = wring[wslot, pl.ds(half * 64 + base, 16)]
            for e in range(16):
                wv = jnp.full((16,), w16[e], jnp.float32)
                for j in range(8):
                    rows[base + e, pl.ds(16 * j, 16)] = (
                        rows[base + e, pl.ds(16 * j, 16)] * wv)
            return 0
        lax.fori_loop(0, 4, grp, 0)

    for k in range(cpc):
        if nch == 4:
            chunk = cid * cpc + k
            off = chunk * NP
            orow_base = chunk * NP + stripe0
        else:
            off = 0
            orow_base = cid * NP + stripe0
        offv = jnp.full((16,), off, jnp.int32)

        def zrow(r, _):
            for j in range(8):
                obuf[r, pl.ds(16 * j, 16)] = zero16
            return 0
        lax.fori_loop(0, 16, zrow, 0)

        def zloop(i, _):
            pltpu.sync_copy(obuf, acc.at[pl.ds(stripe0 + i * 16, 16)])
            return 0
        lax.fori_loop(0, STRIPE // 16, zloop, 0)
        plsc.subcore_barrier()

        fetch_w(0, 0, ws0)
        gather(0, 0, rows0, g0, offv)

        def body(m, _):
            r0 = 2 * m

            def do_row(row, wslot, wsem_pair):
                nxt_wslot = 1 - wslot
                nxt_wsem = wsem_pair[1]
                cur_wsem = wsem_pair[0]

                @pl.when(row + 1 < nrows)
                def _():
                    fetch_w(row + 1, nxt_wslot, nxt_wsem)

                @pl.when(row > 0)
                def _():
                    wait_scatter(rows1, s1)
                gather(row, 1, rows1, g1, offv)

                wait_w(wslot, cur_wsem)
                wait_gather(rows0, g0)
                scale(rows0, wslot, 0)
                scatter(row, 0, rows0, s0)

                wait_scatter(rows0, s0)

                @pl.when(row + 1 < nrows)
                def _():
                    gather(row + 1, 0, rows0, g0, offv)

                wait_gather(rows1, g1)
                scale(rows1, wslot, 1)
                scatter(row, 1, rows1, s1)

            do_row(r0, 0, (ws0, ws1))
            do_row(r0 + 1, 1, (ws1, ws0))
            return 0
        lax.fori_loop(0, nrows // 2, body, 0)
        wait_scatter(rows1, s1)
        plsc.subcore_barrier()

        def oloop(i, _):
            pltpu.sync_copy(acc.at[pl.ds(stripe0 + i * 16, 16)], obuf)
            pltpu.sync_copy(obuf, out_ref.at[pl.ds(orow_base + i * 16, 16)])
            return 0
        lax.fori_loop(0, STRIPE // 16, oloop, 0)
        plsc.subcore_barrier()


def _make_spmm(nch):
    ept = EP // NS if nch == 4 else EP // NT
    nrows = ept // 128
    nout = nch if nch == 4 else 2
    return pl.kernel(
        functools.partial(_spmm_body, nch),
        out_type=jax.ShapeDtypeStruct((nout * NP, 128), jnp.float32),
        mesh=_mesh,
        scratch_types=[
            pltpu.VMEM_SHARED((NP, 128), jnp.float32),
            pltpu.VMEM((nrows, 128), jnp.int32),
            pltpu.VMEM((nrows, 128), jnp.int32),
            pltpu.VMEM((2, 128), jnp.float32),
            pltpu.VMEM((64, 128), jnp.float32),
            pltpu.VMEM((64, 128), jnp.float32),
            pltpu.VMEM((16, 128), jnp.float32),
            pltpu.SemaphoreType.DMA,
            pltpu.SemaphoreType.DMA,
            pltpu.SemaphoreType.DMA,
            pltpu.SemaphoreType.DMA,
            pltpu.SemaphoreType.DMA,
            pltpu.SemaphoreType.DMA,
        ],
    )


_spmm4 = _make_spmm(4)
_spmm1c = _make_spmm(1)

_PNW = NP * L // 128 // NT
_PNODES = NP // NT


def _pool_sum(rows, base, sl):
    vs = [rows[base + r, sl] for r in range(L)]
    while len(vs) > 1:
        vs = [vs[i] + vs[i + 1] for i in range(0, len(vs) - 1, 2)] + (
            [vs[-1]] if len(vs) % 2 else [])
    return vs[0]


def _pool_body(fi_ref, emb_ref, out_ref, idxb, rows0, rows1, obuf,
               sem0, sem1):
    cid = lax.axis_index("c")
    sid = lax.axis_index("s")
    tid = cid * NS + sid
    pltpu.sync_copy(fi_ref.at[pl.ds(tid * _PNW, _PNW)], idxb)
    orow0 = tid * _PNODES

    def process(rows, half):
        def node(g, _):
            def col(j, _):
                sl = pl.ds(16 * j, 16)
                obuf[half + g, sl] = _pool_sum(rows, g * L, sl)
                return 0
            lax.fori_loop(0, NFEAT // 16, col, 0)
            return 0
        lax.fori_loop(0, 128 // L, node, 0)

    pltpu.async_copy(emb_ref.at[idxb.at[0]], rows0, sem0).wait()

    def step(k, _):
        w0 = k * 2
        pltpu.async_copy(emb_ref.at[idxb.at[w0 + 1]], rows1, sem1)
        process(rows0, 0)
        pltpu.make_async_copy(emb_ref.at[idxb.at[w0 + 1]], rows1,
                              sem1).wait()

        @pl.when(w0 + 2 < _PNW)
        def _():
            pltpu.async_copy(emb_ref.at[idxb.at[w0 + 2]], rows0, sem0)
        process(rows1, 4)

        pltpu.sync_copy(obuf, out_ref.at[pl.ds(orow0 + k * 8, 8)])

        @pl.when(w0 + 2 < _PNW)
        def _():
            pltpu.make_async_copy(emb_ref.at[idxb.at[w0 + 2]], rows0,
                                  sem0).wait()
        return 0
    lax.fori_loop(0, _PNW // 2, step, 0)


_pool = pl.kernel(
    _pool_body,
    out_type=jax.ShapeDtypeStruct((NP, NFEAT), jnp.float32),
    mesh=_mesh,
    scratch_types=[
        pltpu.VMEM((_PNW, 128), jnp.int32),
        pltpu.VMEM((128, NFEAT), jnp.float32),
        pltpu.VMEM((128, NFEAT), jnp.float32),
        pltpu.VMEM((8, NFEAT), jnp.float32),
        pltpu.SemaphoreType.DMA,
        pltpu.SemaphoreType.DMA,
    ],
)


def _mm1_body(x_ref, w_ref, o_ref):
    o_ref[0] = jnp.dot(x_ref[...], w_ref[...],
                       preferred_element_type=jnp.float32) * (1.0 / L)


def _mm1(x, w):
    return pl.pallas_call(
        _mm1_body,
        grid=(NP // _M_BLK, NHID // 128),
        in_specs=[
            pl.BlockSpec((_M_BLK, NFEAT), lambda i, c: (i, 0)),
            pl.BlockSpec((NFEAT, 128), lambda i, c: (0, c)),
        ],
        out_specs=pl.BlockSpec((1, _M_BLK, 128), lambda i, c: (c, i, 0)),
        out_shape=jax.ShapeDtypeStruct((NHID // 128, NP, 128), jnp.float32),
    )(x, w)


def _mm2_body(x_ref, b_ref, w_ref, o_ref):
    c = pl.program_id(1)
    h = jnp.maximum(x_ref[0] + b_ref[0], 0.0)
    p = jnp.dot(h, w_ref[0], preferred_element_type=jnp.float32)

    @pl.when(c == 0)
    def _():
        o_ref[...] = p

    @pl.when(c > 0)
    def _():
        o_ref[...] += p


def _mm2(x, b, w):
    return pl.pallas_call(
        _mm2_body,
        grid=(NP // _M_BLK, NHID // 128),
        in_specs=[
            pl.BlockSpec((1, _M_BLK, 128), lambda i, c: (c, i, 0)),
            pl.BlockSpec((1, 1, 128), lambda i, c: (c, 0, 0)),
            pl.BlockSpec((1, 128, NCLASS), lambda i, c: (c, 0, 0)),
        ],
        out_specs=pl.BlockSpec((_M_BLK, NCLASS), lambda i, c: (i, 0)),
        out_shape=jax.ShapeDtypeStruct((NP, NCLASS), jnp.float32),
    )(x, b.reshape(NHID // 128, 1, 128), w.reshape(NHID // 128, 128, NCLASS))


def kernel(x_index, features_index, edge_index, edge_weight, embedding, W1, b1, W2, b2):
    src = edge_index[0]
    dst = edge_index[1]
    pad = EP - E
    fill = (jnp.arange(pad, dtype=jnp.int32) % N)
    src2 = jnp.concatenate([src, fill]).reshape(EP // 128, 128)
    dst2 = jnp.concatenate([dst, fill]).reshape(EP // 128, 128)
    w2 = jnp.concatenate(
        [edge_weight, jnp.zeros((pad,), jnp.float32)]).reshape(EP // 128, 128)

    fi_fill = (jnp.arange((NP - N) * L, dtype=jnp.int32)
               % embedding.shape[0]).reshape(NP - N, L)
    fi2 = jnp.concatenate([features_index, fi_fill]).reshape(NP * L // 128, 128)
    pooled = _pool(fi2, embedding)

    support = _mm1(pooled, W1)
    agg1 = _spmm4(support.reshape(4 * NP, 128), src2, dst2, w2)

    y = _mm2(agg1.reshape(4, NP, 128), b1, W2)

    agg2 = _spmm1c(y, src2, dst2, w2)

    xi = x_index.astype(jnp.int32)
    return (jnp.take(agg2, xi, axis=0, mode="clip")
            + jnp.take(agg2, NP + xi, axis=0, mode="clip") + b2)
